# baseline (speedup 1.0000x reference)
import functools

import jax
import jax.numpy as jnp
import numpy as np
from jax import lax
from jax.experimental import pallas as pl
from jax.experimental.pallas import tpu as pltpu
from jax.experimental.pallas import tpu_sc as plsc

B, T, C, E, K = 2, 2048, 768, 8, 2
CF = 1.25
CAP = int(T / E * CF)
H4 = 4 * C
EBC = E * B * CAP
NC, NS = 2, 16
NW = NC * NS

_SEL_TABLE = np.stack(
    [
        np.linspace(0, n - 1, CAP).astype(np.int64)
        if n > CAP
        else np.zeros(CAP, np.int64)
        for n in range(T + 1)
    ]
).astype(np.float32)
_SEL_PAD = np.zeros((2056, 384), np.float32)
_SEL_PAD[: T + 1, :CAP] = _SEL_TABLE
_SEL_PAD_T = np.ascontiguousarray(_SEL_PAD.T)

_TRI = np.triu(np.ones((T, T), np.float32), 1).astype(jnp.bfloat16)


def _router_body(x_ref, w1_ref, b1_ref, w2_ref, b2_ref, w3_ref, b3_ref,
                 tp_ref, te_ref):
    xb = x_ref[...]
    h = jax.nn.relu(jnp.dot(xb, w1_ref[...],
                            preferred_element_type=jnp.float32) + b1_ref[...])
    h = jax.nn.relu(jnp.dot(h, w2_ref[...],
                            preferred_element_type=jnp.float32) + b2_ref[...])
    logits = jnp.dot(h, w3_ref[...],
                     preferred_element_type=jnp.float32) + b3_ref[...]
    p = jax.nn.softmax(logits, axis=-1)
    m1 = jnp.max(p, axis=-1)
    a1 = jnp.argmax(p, axis=-1).astype(jnp.int32)
    cols = jax.lax.broadcasted_iota(jnp.int32, p.shape, 1)
    pm = jnp.where(cols == a1[:, None], jnp.float32(-1.0), p)
    m2 = jnp.max(pm, axis=-1)
    a2 = jnp.argmax(pm, axis=-1).astype(jnp.int32)
    tp_ref[0, 0, :] = m1
    tp_ref[0, 1, :] = m2
    te_ref[0, 0, :] = a1
    te_ref[0, 1, :] = a2


def _router(x2d, Wr1, br1, Wr2, br2, Wr3, br3):
    TB = 256
    n_tb = (B * T) // TB
    per_b = T // TB
    out_shape = [
        jax.ShapeDtypeStruct((B, K, T), jnp.float32),
        jax.ShapeDtypeStruct((B, K, T), jnp.int32),
    ]
    in_specs = [
        pl.BlockSpec((TB, C), lambda i: (i, 0)),
        pl.BlockSpec((C, H4), lambda i: (0, 0)),
        pl.BlockSpec((1, H4), lambda i: (0, 0)),
        pl.BlockSpec((H4, H4), lambda i: (0, 0)),
        pl.BlockSpec((1, H4), lambda i: (0, 0)),
        pl.BlockSpec((H4, E), lambda i: (0, 0)),
        pl.BlockSpec((1, E), lambda i: (0, 0)),
    ]
    out_specs = [
        pl.BlockSpec((1, K, TB), lambda i: (i // per_b, 0, i % per_b)),
        pl.BlockSpec((1, K, TB), lambda i: (i // per_b, 0, i % per_b)),
    ]
    return pl.pallas_call(
        _router_body, grid=(n_tb,), in_specs=in_specs, out_specs=out_specs,
        out_shape=out_shape,
    )(x2d, Wr1, br1.reshape(1, H4), Wr2, br2.reshape(1, H4), Wr3,
      br3.reshape(1, E))


def _plan_body(te_ref, tp_ref, selt_ref, tri_ref, gidx_ref, wsel_ref,
               src_ref, kept_ref):
    b_id = pl.program_id(0)
    e_id = pl.program_id(1)
    teb = te_ref[0]
    tpb = tp_ref[0]
    m = teb == e_id
    m0 = m[0:1, :]
    m1 = m[1:2, :]
    anym = jnp.logical_or(m0, m1)
    af = anym.astype(jnp.float32)

    rank_row = jnp.dot(af.astype(jnp.bfloat16), tri_ref[...],
                       preferred_element_type=jnp.float32)

    count = jnp.sum(af)
    count_i = count.astype(jnp.int32)
    oh_cnt = (jax.lax.broadcasted_iota(jnp.int32, (1, 2056), 1)
              == count_i).astype(jnp.float32)
    sel_col = jax.lax.dot_general(
        selt_ref[...], oh_cnt, (((1,), (1,)), ((), ())),
        precision=jax.lax.Precision.HIGHEST,
        preferred_element_type=jnp.float32)[:CAP, :]
    jcap = jax.lax.broadcasted_iota(jnp.int32, (CAP, 1), 0)
    jcap = jcap.astype(jnp.float32)
    over = count > jnp.float32(CAP)
    sel_eff = jnp.where(over, sel_col, jcap)
    validj = jnp.logical_or(over, jcap < count)

    oht = jnp.logical_and(sel_eff == rank_row, anym)
    oht = jnp.logical_and(oht, validj).astype(jnp.float32)

    iota_t = jax.lax.broadcasted_iota(jnp.int32, (T, 1), 0)
    iota_t = iota_t.astype(jnp.float32)
    tsel_col = jnp.dot(oht, iota_t, precision=jax.lax.Precision.HIGHEST,
                       preferred_element_type=jnp.float32)
    p_slot = jnp.where(m0, tpb[0:1, :], tpb[1:2, :])
    wsel_col = jax.lax.dot_general(oht, p_slot, (((1,), (1,)), ((), ())),
                                   precision=jax.lax.Precision.HIGHEST,
                                   preferred_element_type=jnp.float32)

    gidx_ref[0, 0] = (tsel_col + (b_id * T).astype(jnp.float32)
                      + 0.5).astype(jnp.int32)
    wsel_ref[0, 0] = wsel_col

    jrow = jax.lax.broadcasted_iota(jnp.int32, (1, CAP), 1)
    jrow = jrow.astype(jnp.float32)
    j_row = jnp.dot(jrow, oht, precision=jax.lax.Precision.HIGHEST,
                    preferred_element_type=jnp.float32)
    kept_row = jnp.sum(oht, axis=0, keepdims=True)
    base = ((e_id * B + b_id) * CAP).astype(jnp.float32)
    val = j_row + base
    c0 = jnp.where(jnp.logical_and(m0, kept_row > 0), val, 0.0)
    c1 = jnp.where(jnp.logical_and(m1, kept_row > 0), val, 0.0)
    k0 = jnp.where(jnp.logical_and(m0, kept_row > 0), 1.0, 0.0)
    k1 = jnp.where(jnp.logical_and(m1, kept_row > 0), 1.0, 0.0)
    src_c = jnp.concatenate([c0, c1], axis=0)
    kept_c = jnp.concatenate([k0, k1], axis=0)

    @pl.when(e_id == 0)
    def _():
        src_ref[0] = src_c
        kept_ref[0] = kept_c

    @pl.when(e_id != 0)
    def _():
        src_ref[0] = src_ref[0] + src_c
        kept_ref[0] = kept_ref[0] + kept_c


def _plan(te, tp, selt, tri):
    out_shape = [
        jax.ShapeDtypeStruct((E, B, CAP, 1), jnp.int32),
        jax.ShapeDtypeStruct((E, B, CAP, 1), jnp.float32),
        jax.ShapeDtypeStruct((B, K, T), jnp.float32),
        jax.ShapeDtypeStruct((B, K, T), jnp.float32),
    ]
    in_specs = [
        pl.BlockSpec((1, K, T), lambda b, e: (b, 0, 0)),
        pl.BlockSpec((1, K, T), lambda b, e: (b, 0, 0)),
        pl.BlockSpec((384, 2056), lambda b, e: (0, 0)),
        pl.BlockSpec((T, T), lambda b, e: (0, 0)),
    ]
    out_specs = [
        pl.BlockSpec((1, 1, CAP, 1), lambda b, e: (e, b, 0, 0)),
        pl.BlockSpec((1, 1, CAP, 1), lambda b, e: (e, b, 0, 0)),
        pl.BlockSpec((1, K, T), lambda b, e: (b, 0, 0)),
        pl.BlockSpec((1, K, T), lambda b, e: (b, 0, 0)),
    ]
    return pl.pallas_call(
        _plan_body, grid=(B, E), in_specs=in_specs, out_specs=out_specs,
        out_shape=out_shape,
    )(te, tp, selt, tri)


_GCH = EBC // NW // 2


def _sc_gather(x2d, gidx):
    mesh = plsc.VectorSubcoreMesh(core_axis_name="c", subcore_axis_name="s",
                                  num_cores=NC, num_subcores=NS)

    @functools.partial(
        pl.kernel, mesh=mesh,
        out_type=jax.ShapeDtypeStruct((EBC, C), jnp.float32),
        scratch_types=[
            pltpu.VMEM((_GCH,), jnp.int32),
            pltpu.VMEM((_GCH, C), jnp.float32),
            pltpu.SemaphoreType.DMA,
        ],
    )
    def k(x_hbm, gidx_hbm, out_hbm, idx_v, rows_v, sem):
        wid = lax.axis_index("s") * NC + lax.axis_index("c")
        base = wid * (EBC // NW)
        for cch in range(2):
            off = base + cch * _GCH
            pltpu.sync_copy(gidx_hbm.at[pl.ds(off, _GCH)], idx_v)
            pltpu.async_copy(x_hbm.at[idx_v], rows_v, sem).wait()
            pltpu.sync_copy(rows_v, out_hbm.at[pl.ds(off, _GCH)])

    return k(x2d, gidx)


def _ffn_body(xg_ref, w1_ref, b1_ref, w2_ref, b2_ref, ws_ref, og_ref):
    xg = xg_ref[0]
    og = b2_ref[0]
    for c in range(2):
        sl = slice(c * (H4 // 2), (c + 1) * (H4 // 2))
        hc = jax.nn.relu(jnp.dot(xg, w1_ref[0, :, sl],
                                 preferred_element_type=jnp.float32)
                         + b1_ref[0, :, sl])
        og = og + jnp.dot(hc, w2_ref[0, sl, :],
                          preferred_element_type=jnp.float32)
    og_ref[0] = og * ws_ref[0]


def _ffn(xg3, W1, b1, W2, b2, ws3):
    BC = B * CAP
    in_specs = [
        pl.BlockSpec((1, BC, C), lambda e: (e, 0, 0)),
        pl.BlockSpec((1, C, H4), lambda e: (e, 0, 0)),
        pl.BlockSpec((1, 1, H4), lambda e: (e, 0, 0)),
        pl.BlockSpec((1, H4, C), lambda e: (e, 0, 0)),
        pl.BlockSpec((1, 1, C), lambda e: (e, 0, 0)),
        pl.BlockSpec((1, BC, 1), lambda e: (e, 0, 0)),
    ]
    out_specs = pl.BlockSpec((1, BC, C), lambda e: (e, 0, 0))
    return pl.pallas_call(
        _ffn_body, grid=(E,), in_specs=in_specs, out_specs=out_specs,
        out_shape=jax.ShapeDtypeStruct((E, BC, C), jnp.float32),
    )(xg3, W1, b1.reshape(E, 1, H4), W2, b2.reshape(E, 1, C), ws3)


_TPW = (B * T) // NW
_TCH = _TPW // 2


def _sc_combine(ogwp, src0, src1):
    mesh = plsc.VectorSubcoreMesh(core_axis_name="c", subcore_axis_name="s",
                                  num_cores=NC, num_subcores=NS)

    @functools.partial(
        pl.kernel, mesh=mesh,
        out_type=jax.ShapeDtypeStruct((B * T, C), jnp.float32),
        scratch_types=[
            pltpu.VMEM((_TCH,), jnp.int32),
            pltpu.VMEM((_TCH,), jnp.int32),
            pltpu.VMEM((_TCH, C), jnp.float32),
            pltpu.VMEM((_TCH, C), jnp.float32),
            pltpu.SemaphoreType.DMA,
        ],
    )
    def k(ogw_hbm, s0_hbm, s1_hbm, out_hbm, i0_v, i1_v, g0_v, g1_v, sem):
        wid = lax.axis_index("s") * NC + lax.axis_index("c")
        tbase = wid * _TPW
        for half in range(2):
            off = tbase + half * _TCH
            pltpu.sync_copy(s0_hbm.at[pl.ds(off, _TCH)], i0_v)
            pltpu.sync_copy(s1_hbm.at[pl.ds(off, _TCH)], i1_v)
            pltpu.async_copy(ogw_hbm.at[i0_v], g0_v, sem).wait()
            pltpu.async_copy(ogw_hbm.at[i1_v], g1_v, sem).wait()

            def body(r, _):
                for c in range(C // 16):
                    sl = pl.ds(c * 16, 16)
                    g0_v[r, sl] = g0_v[r, sl] + g1_v[r, sl]
                return _

            lax.fori_loop(0, _TCH, body, 0)
            pltpu.sync_copy(g0_v, out_hbm.at[pl.ds(off, _TCH)])

    return k(ogwp, src0, src1)


@jax.jit
def kernel(x, Wr1, br1, Wr2, br2, Wr3, br3, W1, b1, W2, b2):
    x2d = x.reshape(B * T, C)
    tp, te = _router(x2d, Wr1, br1, Wr2, br2, Wr3, br3)
    selt = jnp.asarray(_SEL_PAD_T)
    tri = jnp.asarray(_TRI)
    gidx4, wsel4, src, kept = _plan(te, tp, selt, tri)
    xg = _sc_gather(x2d, gidx4.reshape(EBC))
    ogw = _ffn(xg.reshape(E, B * CAP, C), W1, b1, W2, b2,
               wsel4.reshape(E, B * CAP, 1))
    srci = (src + (1.0 - kept) * EBC + 0.5).astype(jnp.int32)
    ogwp = jnp.concatenate(
        [ogw.reshape(EBC, C), jnp.zeros((8, C), jnp.float32)], axis=0)
    y2d = _sc_combine(ogwp, srci[:, 0, :].reshape(B * T),
                      srci[:, 1, :].reshape(B * T))
    return y2d.reshape(B, T, C)


# speedup vs baseline: 1.0056x; 1.0056x over previous
import functools

import jax
import jax.numpy as jnp
import numpy as np
from jax import lax
from jax.experimental import pallas as pl
from jax.experimental.pallas import tpu as pltpu
from jax.experimental.pallas import tpu_sc as plsc

B, T, C, E, K = 2, 2048, 768, 8, 2
CF = 1.25
CAP = int(T / E * CF)
H4 = 4 * C
EBC = E * B * CAP
NC, NS = 2, 16
NW = NC * NS

_SEL_TABLE = np.stack(
    [
        np.linspace(0, n - 1, CAP).astype(np.int64)
        if n > CAP
        else np.zeros(CAP, np.int64)
        for n in range(T + 1)
    ]
).astype(np.float32)
_SEL_PAD = np.zeros((2056, 384), np.float32)
_SEL_PAD[: T + 1, :CAP] = _SEL_TABLE
_SEL_PAD_T = np.ascontiguousarray(_SEL_PAD.T)

_TRI = np.triu(np.ones((T, T), np.float32), 1).astype(jnp.bfloat16)


def _router_body(x_ref, w1_ref, b1_ref, w2_ref, b2_ref, w3_ref, b3_ref,
                 tp_ref, te_ref):
    xb = x_ref[...]
    h = jax.nn.relu(jnp.dot(xb, w1_ref[...],
                            preferred_element_type=jnp.float32) + b1_ref[...])
    h = jax.nn.relu(jnp.dot(h, w2_ref[...],
                            preferred_element_type=jnp.float32) + b2_ref[...])
    logits = jnp.dot(h, w3_ref[...],
                     preferred_element_type=jnp.float32) + b3_ref[...]
    p = jax.nn.softmax(logits, axis=-1)
    m1 = jnp.max(p, axis=-1)
    a1 = jnp.argmax(p, axis=-1).astype(jnp.int32)
    cols = jax.lax.broadcasted_iota(jnp.int32, p.shape, 1)
    pm = jnp.where(cols == a1[:, None], jnp.float32(-1.0), p)
    m2 = jnp.max(pm, axis=-1)
    a2 = jnp.argmax(pm, axis=-1).astype(jnp.int32)
    tp_ref[0, 0, :] = m1
    tp_ref[0, 1, :] = m2
    te_ref[0, 0, :] = a1
    te_ref[0, 1, :] = a2


def _router(x2d, Wr1, br1, Wr2, br2, Wr3, br3):
    TB = 256
    n_tb = (B * T) // TB
    per_b = T // TB
    out_shape = [
        jax.ShapeDtypeStruct((B, K, T), jnp.float32),
        jax.ShapeDtypeStruct((B, K, T), jnp.int32),
    ]
    in_specs = [
        pl.BlockSpec((TB, C), lambda i: (i, 0)),
        pl.BlockSpec((C, H4), lambda i: (0, 0)),
        pl.BlockSpec((1, H4), lambda i: (0, 0)),
        pl.BlockSpec((H4, H4), lambda i: (0, 0)),
        pl.BlockSpec((1, H4), lambda i: (0, 0)),
        pl.BlockSpec((H4, E), lambda i: (0, 0)),
        pl.BlockSpec((1, E), lambda i: (0, 0)),
    ]
    out_specs = [
        pl.BlockSpec((1, K, TB), lambda i: (i // per_b, 0, i % per_b)),
        pl.BlockSpec((1, K, TB), lambda i: (i // per_b, 0, i % per_b)),
    ]
    return pl.pallas_call(
        _router_body, grid=(n_tb,), in_specs=in_specs, out_specs=out_specs,
        out_shape=out_shape,
    )(x2d, Wr1, br1.reshape(1, H4), Wr2, br2.reshape(1, H4), Wr3,
      br3.reshape(1, E))


def _plan_body(te_ref, tp_ref, selt_ref, tri_ref, gidx_ref, wsel_ref,
               src_ref, kept_ref):
    b_id = pl.program_id(0)
    e_id = pl.program_id(1)
    teb = te_ref[0]
    tpb = tp_ref[0]
    m = teb == e_id
    m0 = m[0:1, :]
    m1 = m[1:2, :]
    anym = jnp.logical_or(m0, m1)
    af = anym.astype(jnp.float32)

    rank_row = jnp.dot(af.astype(jnp.bfloat16), tri_ref[...],
                       preferred_element_type=jnp.float32)

    count = jnp.sum(af)
    count_i = count.astype(jnp.int32)
    oh_cnt = (jax.lax.broadcasted_iota(jnp.int32, (1, 2056), 1)
              == count_i).astype(jnp.float32)
    sel_col = jax.lax.dot_general(
        selt_ref[...], oh_cnt, (((1,), (1,)), ((), ())),
        precision=jax.lax.Precision.HIGHEST,
        preferred_element_type=jnp.float32)[:CAP, :]
    jcap = jax.lax.broadcasted_iota(jnp.int32, (CAP, 1), 0)
    jcap = jcap.astype(jnp.float32)
    over = count > jnp.float32(CAP)
    sel_eff = jnp.where(over, sel_col, jcap)
    validj = jnp.logical_or(over, jcap < count)

    oht = jnp.logical_and(sel_eff == rank_row, anym)
    oht = jnp.logical_and(oht, validj).astype(jnp.float32)

    iota_t = jax.lax.broadcasted_iota(jnp.int32, (T, 1), 0)
    iota_t = iota_t.astype(jnp.float32)
    tsel_col = jnp.dot(oht, iota_t, precision=jax.lax.Precision.HIGHEST,
                       preferred_element_type=jnp.float32)
    p_slot = jnp.where(m0, tpb[0:1, :], tpb[1:2, :])
    wsel_col = jax.lax.dot_general(oht, p_slot, (((1,), (1,)), ((), ())),
                                   precision=jax.lax.Precision.HIGHEST,
                                   preferred_element_type=jnp.float32)

    gidx_ref[0, 0] = (tsel_col + (b_id * T).astype(jnp.float32)
                      + 0.5).astype(jnp.int32)
    wsel_ref[0, 0] = wsel_col

    jrow = jax.lax.broadcasted_iota(jnp.int32, (1, CAP), 1)
    jrow = jrow.astype(jnp.float32)
    j_row = jnp.dot(jrow, oht, precision=jax.lax.Precision.HIGHEST,
                    preferred_element_type=jnp.float32)
    kept_row = jnp.sum(oht, axis=0, keepdims=True)
    base = ((e_id * B + b_id) * CAP).astype(jnp.float32)
    val = j_row + base
    c0 = jnp.where(jnp.logical_and(m0, kept_row > 0), val, 0.0)
    c1 = jnp.where(jnp.logical_and(m1, kept_row > 0), val, 0.0)
    k0 = jnp.where(jnp.logical_and(m0, kept_row > 0), 1.0, 0.0)
    k1 = jnp.where(jnp.logical_and(m1, kept_row > 0), 1.0, 0.0)
    src_c = jnp.concatenate([c0, c1], axis=0)
    kept_c = jnp.concatenate([k0, k1], axis=0)

    @pl.when(e_id == 0)
    def _():
        src_ref[0] = src_c
        kept_ref[0] = kept_c

    @pl.when(e_id != 0)
    def _():
        src_ref[0] = src_ref[0] + src_c
        kept_ref[0] = kept_ref[0] + kept_c


def _plan(te, tp, selt, tri):
    out_shape = [
        jax.ShapeDtypeStruct((E, B, CAP, 1), jnp.int32),
        jax.ShapeDtypeStruct((E, B, CAP, 1), jnp.float32),
        jax.ShapeDtypeStruct((B, K, T), jnp.float32),
        jax.ShapeDtypeStruct((B, K, T), jnp.float32),
    ]
    in_specs = [
        pl.BlockSpec((1, K, T), lambda b, e: (b, 0, 0)),
        pl.BlockSpec((1, K, T), lambda b, e: (b, 0, 0)),
        pl.BlockSpec((384, 2056), lambda b, e: (0, 0)),
        pl.BlockSpec((T, T), lambda b, e: (0, 0)),
    ]
    out_specs = [
        pl.BlockSpec((1, 1, CAP, 1), lambda b, e: (e, b, 0, 0)),
        pl.BlockSpec((1, 1, CAP, 1), lambda b, e: (e, b, 0, 0)),
        pl.BlockSpec((1, K, T), lambda b, e: (b, 0, 0)),
        pl.BlockSpec((1, K, T), lambda b, e: (b, 0, 0)),
    ]
    return pl.pallas_call(
        _plan_body, grid=(B, E), in_specs=in_specs, out_specs=out_specs,
        out_shape=out_shape,
    )(te, tp, selt, tri)


_GCH = EBC // NW // 2


def _sc_gather(x2d, gidx):
    mesh = plsc.VectorSubcoreMesh(core_axis_name="c", subcore_axis_name="s",
                                  num_cores=NC, num_subcores=NS)

    @functools.partial(
        pl.kernel, mesh=mesh,
        out_type=jax.ShapeDtypeStruct((EBC, C), jnp.float32),
        scratch_types=[
            pltpu.VMEM((_GCH,), jnp.int32),
            pltpu.VMEM((_GCH,), jnp.int32),
            pltpu.VMEM((_GCH, C), jnp.float32),
            pltpu.VMEM((_GCH, C), jnp.float32),
            pltpu.SemaphoreType.DMA,
        ],
    )
    def k(x_hbm, gidx_hbm, out_hbm, idx_v, idx2_v, rows_v, rows2_v, sem):
        wid = lax.axis_index("s") * NC + lax.axis_index("c")
        base = wid * (EBC // NW)
        pltpu.sync_copy(gidx_hbm.at[pl.ds(base, _GCH)], idx_v)
        pltpu.sync_copy(gidx_hbm.at[pl.ds(base + _GCH, _GCH)], idx2_v)
        c1 = pltpu.async_copy(x_hbm.at[idx_v], rows_v, sem)
        c2 = pltpu.async_copy(x_hbm.at[idx2_v], rows2_v, sem)
        c1.wait()
        pltpu.sync_copy(rows_v, out_hbm.at[pl.ds(base, _GCH)])
        c2.wait()
        pltpu.sync_copy(rows2_v, out_hbm.at[pl.ds(base + _GCH, _GCH)])

    return k(x2d, gidx)


def _ffn_body(xg_ref, w1_ref, b1_ref, w2_ref, b2_ref, ws_ref, og_ref):
    xg = xg_ref[0]
    og = b2_ref[0]
    for c in range(2):
        sl = slice(c * (H4 // 2), (c + 1) * (H4 // 2))
        hc = jax.nn.relu(jnp.dot(xg, w1_ref[0, :, sl],
                                 preferred_element_type=jnp.float32)
                         + b1_ref[0, :, sl])
        og = og + jnp.dot(hc, w2_ref[0, sl, :],
                          preferred_element_type=jnp.float32)
    og_ref[0] = og * ws_ref[0]


def _ffn(xg3, W1, b1, W2, b2, ws3):
    BC = B * CAP
    in_specs = [
        pl.BlockSpec((1, BC, C), lambda e: (e, 0, 0)),
        pl.BlockSpec((1, C, H4), lambda e: (e, 0, 0)),
        pl.BlockSpec((1, 1, H4), lambda e: (e, 0, 0)),
        pl.BlockSpec((1, H4, C), lambda e: (e, 0, 0)),
        pl.BlockSpec((1, 1, C), lambda e: (e, 0, 0)),
        pl.BlockSpec((1, BC, 1), lambda e: (e, 0, 0)),
    ]
    out_specs = pl.BlockSpec((1, BC, C), lambda e: (e, 0, 0))
    return pl.pallas_call(
        _ffn_body, grid=(E,), in_specs=in_specs, out_specs=out_specs,
        out_shape=jax.ShapeDtypeStruct((E, BC, C), jnp.float32),
    )(xg3, W1, b1.reshape(E, 1, H4), W2, b2.reshape(E, 1, C), ws3)


_TPW = (B * T) // NW
_TCH = _TPW // 2


def _sc_combine(ogwp, src0, src1):
    mesh = plsc.VectorSubcoreMesh(core_axis_name="c", subcore_axis_name="s",
                                  num_cores=NC, num_subcores=NS)

    @functools.partial(
        pl.kernel, mesh=mesh,
        out_type=jax.ShapeDtypeStruct((B * T, C), jnp.float32),
        scratch_types=[
            pltpu.VMEM((_TCH,), jnp.int32),
            pltpu.VMEM((_TCH,), jnp.int32),
            pltpu.VMEM((_TCH, C), jnp.float32),
            pltpu.VMEM((_TCH, C), jnp.float32),
            pltpu.SemaphoreType.DMA,
        ],
    )
    def k(ogw_hbm, s0_hbm, s1_hbm, out_hbm, i0_v, i1_v, g0_v, g1_v, sem):
        wid = lax.axis_index("s") * NC + lax.axis_index("c")
        tbase = wid * _TPW
        for half in range(2):
            off = tbase + half * _TCH
            pltpu.sync_copy(s0_hbm.at[pl.ds(off, _TCH)], i0_v)
            pltpu.sync_copy(s1_hbm.at[pl.ds(off, _TCH)], i1_v)
            c0 = pltpu.async_copy(ogw_hbm.at[i0_v], g0_v, sem)
            c1 = pltpu.async_copy(ogw_hbm.at[i1_v], g1_v, sem)
            c0.wait()
            c1.wait()

            @plsc.parallel_loop(0, _TCH * (C // 16), 1, unroll=8)
            def body(i):
                r = i // (C // 16)
                sl = pl.ds((i % (C // 16)) * 16, 16)
                g0_v[r, sl] = g0_v[r, sl] + g1_v[r, sl]

            pltpu.sync_copy(g0_v, out_hbm.at[pl.ds(off, _TCH)])

    return k(ogwp, src0, src1)


@jax.jit
def kernel(x, Wr1, br1, Wr2, br2, Wr3, br3, W1, b1, W2, b2):
    x2d = x.reshape(B * T, C)
    tp, te = _router(x2d, Wr1, br1, Wr2, br2, Wr3, br3)
    selt = jnp.asarray(_SEL_PAD_T)
    tri = jnp.asarray(_TRI)
    gidx4, wsel4, src, kept = _plan(te, tp, selt, tri)
    xg = _sc_gather(x2d, gidx4.reshape(EBC))
    ogw = _ffn(xg.reshape(E, B * CAP, C), W1, b1, W2, b2,
               wsel4.reshape(E, B * CAP, 1))
    srci = (src + (1.0 - kept) * EBC + 0.5).astype(jnp.int32)
    ogwp = jnp.concatenate(
        [ogw.reshape(EBC, C), jnp.zeros((8, C), jnp.float32)], axis=0)
    y2d = _sc_combine(ogwp, srci[:, 0, :].reshape(B * T),
                      srci[:, 1, :].reshape(B * T))
    return y2d.reshape(B, T, C)


# speedup vs baseline: 1.0114x; 1.0057x over previous
import functools

import jax
import jax.numpy as jnp
import numpy as np
from jax import lax
from jax.experimental import pallas as pl
from jax.experimental.pallas import tpu as pltpu
from jax.experimental.pallas import tpu_sc as plsc

B, T, C, E, K = 2, 2048, 768, 8, 2
CF = 1.25
CAP = int(T / E * CF)
H4 = 4 * C
EBC = E * B * CAP
NC, NS = 2, 16
NW = NC * NS

_SEL_TABLE = np.stack(
    [
        np.linspace(0, n - 1, CAP).astype(np.int64)
        if n > CAP
        else np.zeros(CAP, np.int64)
        for n in range(T + 1)
    ]
).astype(np.float32)
_SEL_PAD = np.zeros((2056, 384), np.float32)
_SEL_PAD[: T + 1, :CAP] = _SEL_TABLE
_SEL_PAD_T = np.ascontiguousarray(_SEL_PAD.T)

_TRI = np.triu(np.ones((T, T), np.float32), 1).astype(jnp.bfloat16)


def _router_body(x_ref, w1_ref, b1_ref, w2_ref, b2_ref, w3_ref, b3_ref,
                 tp_ref, te_ref):
    xb = x_ref[...]
    h = jax.nn.relu(jnp.dot(xb, w1_ref[...],
                            preferred_element_type=jnp.float32) + b1_ref[...])
    h = jax.nn.relu(jnp.dot(h, w2_ref[...],
                            preferred_element_type=jnp.float32) + b2_ref[...])
    logits = jnp.dot(h, w3_ref[...],
                     preferred_element_type=jnp.float32) + b3_ref[...]
    p = jax.nn.softmax(logits, axis=-1)
    m1 = jnp.max(p, axis=-1)
    a1 = jnp.argmax(p, axis=-1).astype(jnp.int32)
    cols = jax.lax.broadcasted_iota(jnp.int32, p.shape, 1)
    pm = jnp.where(cols == a1[:, None], jnp.float32(-1.0), p)
    m2 = jnp.max(pm, axis=-1)
    a2 = jnp.argmax(pm, axis=-1).astype(jnp.int32)
    tp_ref[0, 0, :] = m1
    tp_ref[0, 1, :] = m2
    te_ref[0, 0, :] = a1
    te_ref[0, 1, :] = a2


def _router(x2d, Wr1, br1, Wr2, br2, Wr3, br3):
    TB = 256
    n_tb = (B * T) // TB
    per_b = T // TB
    out_shape = [
        jax.ShapeDtypeStruct((B, K, T), jnp.float32),
        jax.ShapeDtypeStruct((B, K, T), jnp.int32),
    ]
    in_specs = [
        pl.BlockSpec((TB, C), lambda i: (i, 0)),
        pl.BlockSpec((C, H4), lambda i: (0, 0)),
        pl.BlockSpec((1, H4), lambda i: (0, 0)),
        pl.BlockSpec((H4, H4), lambda i: (0, 0)),
        pl.BlockSpec((1, H4), lambda i: (0, 0)),
        pl.BlockSpec((H4, E), lambda i: (0, 0)),
        pl.BlockSpec((1, E), lambda i: (0, 0)),
    ]
    out_specs = [
        pl.BlockSpec((1, K, TB), lambda i: (i // per_b, 0, i % per_b)),
        pl.BlockSpec((1, K, TB), lambda i: (i // per_b, 0, i % per_b)),
    ]
    return pl.pallas_call(
        _router_body, grid=(n_tb,), in_specs=in_specs, out_specs=out_specs,
        out_shape=out_shape,
    )(x2d, Wr1, br1.reshape(1, H4), Wr2, br2.reshape(1, H4), Wr3,
      br3.reshape(1, E))


def _plan_body(te_ref, tp_ref, selt_ref, tri_ref, gidx_ref, wsel_ref,
               src_ref, kept_ref):
    b_id = pl.program_id(0)
    e_id = pl.program_id(1)
    teb = te_ref[0]
    tpb = tp_ref[0]
    m = teb == e_id
    m0 = m[0:1, :]
    m1 = m[1:2, :]
    anym = jnp.logical_or(m0, m1)
    af = anym.astype(jnp.float32)

    rank_row = jnp.dot(af.astype(jnp.bfloat16), tri_ref[...],
                       preferred_element_type=jnp.float32)

    count = jnp.sum(af)
    count_i = count.astype(jnp.int32)
    oh_cnt = (jax.lax.broadcasted_iota(jnp.int32, (1, 2056), 1)
              == count_i).astype(jnp.float32)
    sel_col = jax.lax.dot_general(
        selt_ref[...], oh_cnt, (((1,), (1,)), ((), ())),
        precision=jax.lax.Precision.HIGHEST,
        preferred_element_type=jnp.float32)[:CAP, :]
    jcap = jax.lax.broadcasted_iota(jnp.int32, (CAP, 1), 0)
    jcap = jcap.astype(jnp.float32)
    over = count > jnp.float32(CAP)
    sel_eff = jnp.where(over, sel_col, jcap)
    validj = jnp.logical_or(over, jcap < count)

    oht = jnp.logical_and(sel_eff == rank_row, anym)
    oht = jnp.logical_and(oht, validj).astype(jnp.float32)

    iota_t = jax.lax.broadcasted_iota(jnp.int32, (T, 1), 0)
    iota_t = iota_t.astype(jnp.float32)
    tsel_col = jnp.dot(oht, iota_t, precision=jax.lax.Precision.HIGHEST,
                       preferred_element_type=jnp.float32)
    p_slot = jnp.where(m0, tpb[0:1, :], tpb[1:2, :])
    wsel_col = jax.lax.dot_general(oht, p_slot, (((1,), (1,)), ((), ())),
                                   precision=jax.lax.Precision.HIGHEST,
                                   preferred_element_type=jnp.float32)

    gidx_ref[0, 0] = (tsel_col + (b_id * T).astype(jnp.float32)
                      + 0.5).astype(jnp.int32)
    wsel_ref[0, 0] = wsel_col

    jrow = jax.lax.broadcasted_iota(jnp.int32, (1, CAP), 1)
    jrow = jrow.astype(jnp.float32)
    j_row = jnp.dot(jrow, oht, precision=jax.lax.Precision.HIGHEST,
                    preferred_element_type=jnp.float32)
    kept_row = jnp.sum(oht, axis=0, keepdims=True)
    base = ((e_id * B + b_id) * CAP).astype(jnp.float32)
    val = j_row + base
    c0 = jnp.where(jnp.logical_and(m0, kept_row > 0), val, 0.0)
    c1 = jnp.where(jnp.logical_and(m1, kept_row > 0), val, 0.0)
    k0 = jnp.where(jnp.logical_and(m0, kept_row > 0), 1.0, 0.0)
    k1 = jnp.where(jnp.logical_and(m1, kept_row > 0), 1.0, 0.0)
    src_c = jnp.concatenate([c0, c1], axis=0)
    kept_c = jnp.concatenate([k0, k1], axis=0)

    @pl.when(e_id == 0)
    def _():
        src_ref[0] = src_c
        kept_ref[0] = kept_c

    @pl.when(e_id != 0)
    def _():
        src_ref[0] = src_ref[0] + src_c
        kept_ref[0] = kept_ref[0] + kept_c


def _plan(te, tp, selt, tri):
    out_shape = [
        jax.ShapeDtypeStruct((E, B, CAP, 1), jnp.int32),
        jax.ShapeDtypeStruct((E, B, CAP, 1), jnp.float32),
        jax.ShapeDtypeStruct((B, K, T), jnp.float32),
        jax.ShapeDtypeStruct((B, K, T), jnp.float32),
    ]
    in_specs = [
        pl.BlockSpec((1, K, T), lambda b, e: (b, 0, 0)),
        pl.BlockSpec((1, K, T), lambda b, e: (b, 0, 0)),
        pl.BlockSpec((384, 2056), lambda b, e: (0, 0)),
        pl.BlockSpec((T, T), lambda b, e: (0, 0)),
    ]
    out_specs = [
        pl.BlockSpec((1, 1, CAP, 1), lambda b, e: (e, b, 0, 0)),
        pl.BlockSpec((1, 1, CAP, 1), lambda b, e: (e, b, 0, 0)),
        pl.BlockSpec((1, K, T), lambda b, e: (b, 0, 0)),
        pl.BlockSpec((1, K, T), lambda b, e: (b, 0, 0)),
    ]
    return pl.pallas_call(
        _plan_body, grid=(B, E), in_specs=in_specs, out_specs=out_specs,
        out_shape=out_shape,
    )(te, tp, selt, tri)


_GCH = EBC // NW // 2


def _sc_gather(x2d, gidx):
    mesh = plsc.VectorSubcoreMesh(core_axis_name="c", subcore_axis_name="s",
                                  num_cores=NC, num_subcores=NS)

    @functools.partial(
        pl.kernel, mesh=mesh,
        out_type=jax.ShapeDtypeStruct((EBC, C), jnp.float32),
        scratch_types=[
            pltpu.VMEM((_GCH,), jnp.int32),
            pltpu.VMEM((_GCH,), jnp.int32),
            pltpu.VMEM((_GCH, C), jnp.float32),
            pltpu.VMEM((_GCH, C), jnp.float32),
            pltpu.SemaphoreType.DMA,
        ],
    )
    def k(x_hbm, gidx_hbm, out_hbm, idx_v, idx2_v, rows_v, rows2_v, sem):
        wid = lax.axis_index("s") * NC + lax.axis_index("c")
        base = wid * (EBC // NW)
        pltpu.sync_copy(gidx_hbm.at[pl.ds(base, _GCH)], idx_v)
        pltpu.sync_copy(gidx_hbm.at[pl.ds(base + _GCH, _GCH)], idx2_v)
        c1 = pltpu.async_copy(x_hbm.at[idx_v], rows_v, sem)
        c2 = pltpu.async_copy(x_hbm.at[idx2_v], rows2_v, sem)
        c1.wait()
        pltpu.sync_copy(rows_v, out_hbm.at[pl.ds(base, _GCH)])
        c2.wait()
        pltpu.sync_copy(rows2_v, out_hbm.at[pl.ds(base + _GCH, _GCH)])

    return k(x2d, gidx)


def _ffn_body(xg_ref, w1_ref, b1_ref, w2_ref, b2_ref, ws_ref, og_ref):
    xg = xg_ref[0]
    og = b2_ref[0]
    for c in range(2):
        sl = slice(c * (H4 // 2), (c + 1) * (H4 // 2))
        hc = jax.nn.relu(jnp.dot(xg, w1_ref[0, :, sl],
                                 preferred_element_type=jnp.float32)
                         + b1_ref[0, :, sl])
        og = og + jnp.dot(hc, w2_ref[0, sl, :],
                          preferred_element_type=jnp.float32)
    og_ref[0] = og * ws_ref[0]


def _ffn(xg3, W1, b1, W2, b2, ws3):
    BC = B * CAP
    in_specs = [
        pl.BlockSpec((1, BC, C), lambda e: (e, 0, 0)),
        pl.BlockSpec((1, C, H4), lambda e: (e, 0, 0)),
        pl.BlockSpec((1, 1, H4), lambda e: (e, 0, 0)),
        pl.BlockSpec((1, H4, C), lambda e: (e, 0, 0)),
        pl.BlockSpec((1, 1, C), lambda e: (e, 0, 0)),
        pl.BlockSpec((1, BC, 1), lambda e: (e, 0, 0)),
    ]
    out_specs = pl.BlockSpec((1, BC, C), lambda e: (e, 0, 0))
    return pl.pallas_call(
        _ffn_body, grid=(E,), in_specs=in_specs, out_specs=out_specs,
        out_shape=jax.ShapeDtypeStruct((E, BC, C), jnp.float32),
    )(xg3, W1, b1.reshape(E, 1, H4), W2, b2.reshape(E, 1, C), ws3)


_TPW = (B * T) // NW
_TCH = _TPW // 2


def _sc_combine(ogwp, src0, src1):
    mesh = plsc.VectorSubcoreMesh(core_axis_name="c", subcore_axis_name="s",
                                  num_cores=NC, num_subcores=NS)

    @functools.partial(
        pl.kernel, mesh=mesh,
        out_type=jax.ShapeDtypeStruct((B * T, C), jnp.float32),
        scratch_types=[
            pltpu.VMEM((_TCH,), jnp.int32),
            pltpu.VMEM((_TCH,), jnp.int32),
            pltpu.VMEM((_TCH, C), jnp.float32),
            pltpu.VMEM((_TCH, C), jnp.float32),
            pltpu.SemaphoreType.DMA,
        ],
    )
    def k(ogw_hbm, s0_hbm, s1_hbm, out_hbm, i0_v, i1_v, g0_v, g1_v, sem):
        wid = lax.axis_index("s") * NC + lax.axis_index("c")
        tbase = wid * _TPW
        for half in range(2):
            off = tbase + half * _TCH
            pltpu.sync_copy(s0_hbm.at[pl.ds(off, _TCH)], i0_v)
            pltpu.sync_copy(s1_hbm.at[pl.ds(off, _TCH)], i1_v)
            c0 = pltpu.async_copy(ogw_hbm.at[i0_v], g0_v, sem)
            c1 = pltpu.async_copy(ogw_hbm.at[i1_v], g1_v, sem)
            c0.wait()
            c1.wait()

            @plsc.parallel_loop(0, _TCH, 1, unroll=2)
            def body(r):
                for c in range(C // 16):
                    sl = pl.ds(c * 16, 16)
                    g0_v[r, sl] = g0_v[r, sl] + g1_v[r, sl]

            pltpu.sync_copy(g0_v, out_hbm.at[pl.ds(off, _TCH)])

    return k(ogwp, src0, src1)


@jax.jit
def kernel(x, Wr1, br1, Wr2, br2, Wr3, br3, W1, b1, W2, b2):
    x2d = x.reshape(B * T, C)
    tp, te = _router(x2d, Wr1, br1, Wr2, br2, Wr3, br3)
    selt = jnp.asarray(_SEL_PAD_T)
    tri = jnp.asarray(_TRI)
    gidx4, wsel4, src, kept = _plan(te, tp, selt, tri)
    xg = _sc_gather(x2d, gidx4.reshape(EBC))
    ogw = _ffn(xg.reshape(E, B * CAP, C), W1, b1, W2, b2,
               wsel4.reshape(E, B * CAP, 1))
    srci = (src + (1.0 - kept) * EBC + 0.5).astype(jnp.int32)
    ogwp = jnp.concatenate(
        [ogw.reshape(EBC, C), jnp.zeros((8, C), jnp.float32)], axis=0)
    y2d = _sc_combine(ogwp, srci[:, 0, :].reshape(B * T),
                      srci[:, 1, :].reshape(B * T))
    return y2d.reshape(B, T, C)
